# Initial kernel scaffold; baseline (speedup 1.0000x reference)
import jax
import jax.numpy as jnp
from jax.experimental import pallas as pl


def kernel(boxes, scores):
    raise NotImplementedError("write your pallas kernel here")


# speedup vs baseline: 71.8947x; 71.8947x over previous
import jax
import jax.numpy as jnp
from jax.experimental import pallas as pl
from jax.experimental.pallas import tpu as pltpu

N = 5000
C = 5
B = 512
NP = 5120
T = NP // B
NMS_T = 0.3
SCORE_T = 0.05
DETS = 100


def _nms_kernel(sboxes_ref, sscores_ref, kept_ref, keep_scr):
    t = pl.program_id(1)
    colsl = pl.ds(t * B, B)
    bx1 = sboxes_ref[0, 0, colsl]
    by1 = sboxes_ref[0, 1, colsl]
    bx2 = sboxes_ref[0, 2, colsl]
    by2 = sboxes_ref[0, 3, colsl]
    ss = sscores_ref[0, 0, colsl]
    area_c = jnp.maximum(bx2 - bx1, 0.0) * jnp.maximum(by2 - by1, 0.0)

    def iou_rows(rx1, ry1, rx2, ry2):
        ra = jnp.maximum(rx2 - rx1, 0.0) * jnp.maximum(ry2 - ry1, 0.0)
        xx1 = jnp.maximum(rx1[:, None], bx1[None, :])
        yy1 = jnp.maximum(ry1[:, None], by1[None, :])
        xx2 = jnp.minimum(rx2[:, None], bx2[None, :])
        yy2 = jnp.minimum(ry2[:, None], by2[None, :])
        inter = jnp.maximum(xx2 - xx1, 0.0) * jnp.maximum(yy2 - yy1, 0.0)
        return inter / (ra[:, None] + area_c[None, :] - inter + 1e-9)

    def body(u, sup):
        rsl = pl.ds(u * B, B)
        iou = iou_rows(
            sboxes_ref[0, 0, rsl],
            sboxes_ref[0, 1, rsl],
            sboxes_ref[0, 2, rsl],
            sboxes_ref[0, 3, rsl],
        )
        keep_u = keep_scr[0, rsl]
        return sup + jnp.sum(
            jnp.where(iou > NMS_T, keep_u[:, None], 0.0), axis=0
        )

    sup = jax.lax.fori_loop(0, t, body, jnp.zeros((B,), jnp.float32))
    validf = ((ss > SCORE_T) & (sup == 0.0)).astype(jnp.float32)

    iou_tt = iou_rows(bx1, by1, bx2, by2)
    rowi = jax.lax.broadcasted_iota(jnp.int32, (B, B), 0)
    coli = jax.lax.broadcasted_iota(jnp.int32, (B, B), 1)
    supmat = ((iou_tt > NMS_T) & (rowi < coli)).astype(jnp.float32)

    def w_cond(carry):
        return jnp.logical_not(carry[1])

    def w_body(carry):
        x, _ = carry
        s = jnp.sum(supmat * x[:, None], axis=0)
        xn = validf * (s == 0.0).astype(jnp.float32)
        return xn, jnp.all(xn == x)

    x, _ = jax.lax.while_loop(w_cond, w_body, (validf, jnp.bool_(False)))
    keep_scr[0, colsl] = x
    kept_ref[0, 0, :] = ss * x


def _limit_kernel(kept_ref, sboxes_ref, out_ref):
    kept = kept_ref[...]

    def bits_f(b):
        return jax.lax.bitcast_convert_type(b, jnp.float32)

    def bis(i, lohi):
        lo, hi = lohi
        mid = (lo + hi) // 2
        cnt = jnp.sum((kept >= bits_f(mid)).astype(jnp.int32))
        ok = cnt >= DETS
        return jnp.where(ok, mid, lo), jnp.where(ok, hi, mid)

    lo, _ = jax.lax.fori_loop(
        0, 32, bis, (jnp.int32(0), jnp.int32(0x3F800001))
    )
    thresh = bits_f(lo)
    m = ((kept >= thresh) & (kept > 0.0)).astype(jnp.float32)
    out_ref[:, 0, :] = kept * m
    for k in range(4):
        out_ref[:, k + 1, :] = sboxes_ref[:, k, :] * m


def kernel(boxes, scores):
    fg = scores[:, 1:]
    order = jnp.argsort(-fg, axis=0).T
    ssorted = jnp.take_along_axis(fg.T, order, axis=1)
    bsorted = jnp.transpose(boxes[order], (0, 2, 1))
    ssorted = jnp.pad(ssorted, ((0, 0), (0, NP - N)))[:, None, :]
    bsorted = jnp.pad(bsorted, ((0, 0), (0, 0), (0, NP - N)))

    kept = pl.pallas_call(
        _nms_kernel,
        grid=(C, T),
        in_specs=[
            pl.BlockSpec((1, 4, NP), lambda c, t: (c, 0, 0)),
            pl.BlockSpec((1, 1, NP), lambda c, t: (c, 0, 0)),
        ],
        out_specs=pl.BlockSpec((1, 1, B), lambda c, t: (c, 0, t)),
        out_shape=jax.ShapeDtypeStruct((C, 1, NP), jnp.float32),
        scratch_shapes=[pltpu.VMEM((1, NP), jnp.float32)],
        compiler_params=pltpu.CompilerParams(
            dimension_semantics=("arbitrary", "arbitrary")
        ),
    )(bsorted, ssorted)

    outs = pl.pallas_call(
        _limit_kernel,
        in_specs=[
            pl.BlockSpec((C, NP), lambda: (0, 0)),
            pl.BlockSpec((C, 4, NP), lambda: (0, 0, 0)),
        ],
        out_specs=pl.BlockSpec((C, 5, NP), lambda: (0, 0, 0)),
        out_shape=jax.ShapeDtypeStruct((C, 5, NP), jnp.float32),
    )(kept[:, 0, :], bsorted)

    inv = jnp.argsort(order, axis=1)
    outs_t = jnp.transpose(outs, (0, 2, 1))
    return jnp.take_along_axis(outs_t, inv[:, :, None], axis=1)


# speedup vs baseline: 76.9947x; 1.0709x over previous
import jax
import jax.numpy as jnp
from jax.experimental import pallas as pl
from jax.experimental.pallas import tpu as pltpu

N = 5000
C = 5
B = 1024
NP = 5120
T = NP // B
NMS_T = 0.3
SCORE_T = 0.05
DETS = 100


def _nms_kernel(sboxes_ref, sscores_ref, kept_ref, keep_scr):
    t = pl.program_id(1)
    colsl = pl.ds(t * B, B)
    bx1 = sboxes_ref[0, 0, colsl]
    by1 = sboxes_ref[0, 1, colsl]
    bx2 = sboxes_ref[0, 2, colsl]
    by2 = sboxes_ref[0, 3, colsl]
    ss = sscores_ref[0, 0, colsl]
    area_c = jnp.maximum(bx2 - bx1, 0.0) * jnp.maximum(by2 - by1, 0.0)

    def iou_rows(rx1, ry1, rx2, ry2):
        ra = jnp.maximum(rx2 - rx1, 0.0) * jnp.maximum(ry2 - ry1, 0.0)
        xx1 = jnp.maximum(rx1[:, None], bx1[None, :])
        yy1 = jnp.maximum(ry1[:, None], by1[None, :])
        xx2 = jnp.minimum(rx2[:, None], bx2[None, :])
        yy2 = jnp.minimum(ry2[:, None], by2[None, :])
        inter = jnp.maximum(xx2 - xx1, 0.0) * jnp.maximum(yy2 - yy1, 0.0)
        return inter / (ra[:, None] + area_c[None, :] - inter + 1e-9)

    def body(u, sup):
        rsl = pl.ds(u * B, B)
        iou = iou_rows(
            sboxes_ref[0, 0, rsl],
            sboxes_ref[0, 1, rsl],
            sboxes_ref[0, 2, rsl],
            sboxes_ref[0, 3, rsl],
        )
        keep_u = keep_scr[0, rsl]
        return sup + jnp.sum(
            jnp.where(iou > NMS_T, keep_u[:, None], 0.0), axis=0
        )

    sup = jax.lax.fori_loop(0, t, body, jnp.zeros((B,), jnp.float32))
    validf = ((ss > SCORE_T) & (sup == 0.0)).astype(jnp.float32)

    iou_tt = iou_rows(bx1, by1, bx2, by2)
    rowi = jax.lax.broadcasted_iota(jnp.int32, (B, B), 0)
    coli = jax.lax.broadcasted_iota(jnp.int32, (B, B), 1)
    supmat = ((iou_tt > NMS_T) & (rowi < coli)).astype(jnp.float32)

    def w_cond(carry):
        return jnp.logical_not(carry[1])

    def w_body(carry):
        x, _ = carry
        s = jnp.sum(supmat * x[:, None], axis=0)
        xn = validf * (s == 0.0).astype(jnp.float32)
        return xn, jnp.all(xn == x)

    x, _ = jax.lax.while_loop(w_cond, w_body, (validf, jnp.bool_(False)))
    keep_scr[0, colsl] = x
    kept_ref[0, 0, :] = ss * x


def _limit_kernel(kept_ref, sboxes_ref, out_ref):
    kept = kept_ref[...]

    def bits_f(b):
        return jax.lax.bitcast_convert_type(b, jnp.float32)

    def bis(i, lohi):
        lo, hi = lohi
        mid = (lo + hi) // 2
        cnt = jnp.sum((kept >= bits_f(mid)).astype(jnp.int32))
        ok = cnt >= DETS
        return jnp.where(ok, mid, lo), jnp.where(ok, hi, mid)

    lo, _ = jax.lax.fori_loop(
        0, 32, bis, (jnp.int32(0), jnp.int32(0x3F800001))
    )
    thresh = bits_f(lo)
    m = ((kept >= thresh) & (kept > 0.0)).astype(jnp.float32)
    out_ref[:, 0, :] = kept * m
    for k in range(4):
        out_ref[:, k + 1, :] = sboxes_ref[:, k, :] * m


def kernel(boxes, scores):
    fg = scores[:, 1:]
    order = jnp.argsort(-fg, axis=0).T
    ssorted = jnp.take_along_axis(fg.T, order, axis=1)
    bsorted = jnp.transpose(boxes[order], (0, 2, 1))
    ssorted = jnp.pad(ssorted, ((0, 0), (0, NP - N)))[:, None, :]
    bsorted = jnp.pad(bsorted, ((0, 0), (0, 0), (0, NP - N)))

    kept = pl.pallas_call(
        _nms_kernel,
        grid=(C, T),
        in_specs=[
            pl.BlockSpec((1, 4, NP), lambda c, t: (c, 0, 0)),
            pl.BlockSpec((1, 1, NP), lambda c, t: (c, 0, 0)),
        ],
        out_specs=pl.BlockSpec((1, 1, B), lambda c, t: (c, 0, t)),
        out_shape=jax.ShapeDtypeStruct((C, 1, NP), jnp.float32),
        scratch_shapes=[pltpu.VMEM((1, NP), jnp.float32)],
        compiler_params=pltpu.CompilerParams(
            dimension_semantics=("arbitrary", "arbitrary")
        ),
    )(bsorted, ssorted)

    outs = pl.pallas_call(
        _limit_kernel,
        in_specs=[
            pl.BlockSpec((C, NP), lambda: (0, 0)),
            pl.BlockSpec((C, 4, NP), lambda: (0, 0, 0)),
        ],
        out_specs=pl.BlockSpec((C, 5, NP), lambda: (0, 0, 0)),
        out_shape=jax.ShapeDtypeStruct((C, 5, NP), jnp.float32),
    )(kept[:, 0, :], bsorted)

    inv = jnp.argsort(order, axis=1)
    outs_t = jnp.transpose(outs, (0, 2, 1))
    return jnp.take_along_axis(outs_t, inv[:, :, None], axis=1)
